# baseline (speedup 1.0000x reference)
import functools

import jax
import jax.numpy as jnp
from jax import lax
from jax.experimental import pallas as pl
from jax.experimental.pallas import tpu as pltpu
from jax.experimental.pallas import tpu_sc as plsc


def _gather_body(table_hbm, idx_hbm, out_hbm, idx_v, sem, *, nc, b_per_w):
    wid = lax.axis_index("s") * nc + lax.axis_index("c")
    base = wid * b_per_w
    pltpu.sync_copy(idx_hbm.at[pl.ds(base, b_per_w)], idx_v)

    def step(g, _):
        v = idx_v[pl.ds(g * 16, 16)]
        for j in range(16):
            t = v[j]
            pltpu.async_copy(
                table_hbm.at[pl.ds(t, 1)],
                out_hbm.at[pl.ds(base + g * 16 + j, 1)],
                sem,
            )
        return ()

    lax.fori_loop(0, b_per_w // 16, step, ())
    pltpu.make_async_copy(
        table_hbm.at[pl.ds(0, b_per_w)],
        out_hbm.at[pl.ds(base, b_per_w)],
        sem,
    ).wait()


def kernel(positional_encoding, time_steps):
    V, D = positional_encoding.shape
    (B,) = time_steps.shape
    info = plsc.get_sparse_core_info()
    nc, ns = info.num_cores, info.num_subcores
    nw = nc * ns
    b_per_w = B // nw
    mesh = plsc.VectorSubcoreMesh(core_axis_name="c", subcore_axis_name="s")
    run = pl.kernel(
        functools.partial(_gather_body, nc=nc, b_per_w=b_per_w),
        mesh=mesh,
        out_type=jax.ShapeDtypeStruct((B, D), positional_encoding.dtype),
        scratch_types=[
            pltpu.VMEM((b_per_w,), jnp.int32),
            pltpu.SemaphoreType.DMA,
        ],
        compiler_params=pltpu.CompilerParams(use_tc_tiling_on_sc=True),
    )
    return run(positional_encoding, time_steps)


# speedup vs baseline: 2.2694x; 2.2694x over previous
import functools

import jax
import jax.numpy as jnp
from jax import lax
from jax.experimental import pallas as pl
from jax.experimental.pallas import tpu as pltpu
from jax.experimental.pallas import tpu_sc as plsc

_CHUNK = 128
_L = 16


def _gather_body(table_hbm, idx_hbm, out_hbm, idx_v, pid_v, pair_v, rows_v, sem,
                 *, nc, b_per_w, d):
    wid = lax.axis_index("s") * nc + lax.axis_index("c")
    base = wid * b_per_w
    n_chunks = b_per_w // _CHUNK
    pltpu.sync_copy(idx_hbm.at[pl.ds(base, b_per_w)], idx_v)
    for k in range(b_per_w // _L):
        v = idx_v[pl.ds(k * _L, _L)]
        pid_v[pl.ds(k * _L, _L)] = lax.shift_right_logical(v, 1)
    copies = []
    for j in range(n_chunks):
        copies.append(
            pltpu.async_copy(
                table_hbm.at[pid_v.at[pl.ds(j * _CHUNK, _CHUNK)]],
                pair_v.at[pl.ds(j * _CHUNK, _CHUNK)],
                sem,
            )
        )
    for c in copies:
        c.wait()

    lanes = lax.iota(jnp.int32, _L)

    def block(i, _):
        r_vec = lanes + i * _L
        tv = idx_v[pl.ds(i * _L, _L)]
        off = (tv & 1) * d
        r2 = lax.shift_right_logical(r_vec, 1)
        c2base = (r_vec & 1) * d
        for c in range(d):
            vals = plsc.load_gather(pair_v, [r_vec, off + c])
            plsc.store_scatter(rows_v, [r2, c2base + c], vals)
        return _

    lax.fori_loop(0, b_per_w // _L, block, 0)
    pltpu.sync_copy(rows_v, out_hbm.at[pl.ds(wid * (b_per_w // 2), b_per_w // 2)])


def kernel(positional_encoding, time_steps):
    V, D = positional_encoding.shape
    (B,) = time_steps.shape
    info = plsc.get_sparse_core_info()
    nc, ns = info.num_cores, info.num_subcores
    nw = nc * ns
    b_per_w = B // nw
    table2 = positional_encoding.reshape(V // 2, 2 * D)
    mesh = plsc.VectorSubcoreMesh(core_axis_name="c", subcore_axis_name="s")
    run = pl.kernel(
        functools.partial(_gather_body, nc=nc, b_per_w=b_per_w, d=D),
        mesh=mesh,
        out_type=jax.ShapeDtypeStruct((B // 2, 2 * D), positional_encoding.dtype),
        scratch_types=[
            pltpu.VMEM((b_per_w,), jnp.int32),
            pltpu.VMEM((b_per_w,), jnp.int32),
            pltpu.VMEM((b_per_w, 2 * D), jnp.float32),
            pltpu.VMEM((b_per_w // 2, 2 * D), jnp.float32),
            pltpu.SemaphoreType.DMA,
        ],
        compiler_params=pltpu.CompilerParams(
            use_tc_tiling_on_sc=True, needs_layout_passes=False
        ),
    )
    return run(table2, time_steps).reshape(B, D)


# speedup vs baseline: 3.1924x; 1.4067x over previous
import functools

import jax
import jax.numpy as jnp
from jax import lax
from jax.experimental import pallas as pl
from jax.experimental.pallas import tpu as pltpu
from jax.experimental.pallas import tpu_sc as plsc

_CHUNK = 128


def _gather_body(table_hbm, idx_hbm, out_hbm, idx_v, rows_v, sem, *, nc, b_per_w):
    wid = lax.axis_index("s") * nc + lax.axis_index("c")
    base = wid * b_per_w
    n_chunks = b_per_w // _CHUNK
    pltpu.sync_copy(idx_hbm.at[pl.ds(base, b_per_w)], idx_v)
    copies = []
    for j in range(n_chunks):
        copies.append(
            pltpu.async_copy(
                table_hbm.at[idx_v.at[pl.ds(j * _CHUNK, _CHUNK)]],
                rows_v.at[pl.ds(j * _CHUNK, _CHUNK)],
                sem,
            )
        )
    for c in copies:
        c.wait()
    pltpu.sync_copy(rows_v, out_hbm.at[pl.ds(base, b_per_w)])


def kernel(positional_encoding, time_steps):
    V, D = positional_encoding.shape
    (B,) = time_steps.shape
    info = plsc.get_sparse_core_info()
    nc, ns = info.num_cores, info.num_subcores
    nw = nc * ns
    b_per_w = B // nw
    mesh = plsc.VectorSubcoreMesh(core_axis_name="c", subcore_axis_name="s")
    run = pl.kernel(
        functools.partial(_gather_body, nc=nc, b_per_w=b_per_w),
        mesh=mesh,
        out_type=jax.ShapeDtypeStruct((B, D), positional_encoding.dtype),
        scratch_types=[
            pltpu.VMEM((b_per_w,), jnp.int32),
            pltpu.VMEM((b_per_w, D), jnp.float32),
            pltpu.SemaphoreType.DMA,
        ],
        compiler_params=pltpu.CompilerParams(use_tc_tiling_on_sc=False),
    )
    return run(positional_encoding, time_steps)
